# baseline (speedup 1.0000x reference)
import jax
import jax.numpy as jnp
from jax import lax
from jax.experimental import pallas as pl
from jax.experimental.pallas import tpu as pltpu

N = 32768
D_IN, D_MODEL, E, D_OUT = 4, 16, 8, 4
BLOCK = 8192
W_ROWS = 376

_DN_00 = (((0,), (0,)), ((), ()))
_DN_01 = (((0,), (1,)), ((), ()))


def _moe_kernel(x_ref, W_ref, out_ref, WF_s, bias_s):
    f32 = jnp.float32
    dot = lambda a, b: jnp.dot(a, b, preferred_element_type=f32)
    dg = lambda a, b, dn: lax.dot_general(a, b, dn, preferred_element_type=f32)

    Wemb = W_ref[0:4, :]
    Wg = W_ref[8:24, 0:8]
    Wr = W_ref[152:168, :]
    Wc = W_ref[168:184, 0:2]
    Wp = W_ref[184:200, 0:4]
    bembC = W_ref[200:216, 0:1]
    brC = W_ref[344:360, 0:1]
    bcC = W_ref[360:362, 0:1]
    bp = W_ref[368:369, 0:4]

    @pl.when(pl.program_id(0) == 0)
    def _fold():
        WcF = dot(Wemb, Wc)
        bcF = dg(Wc, bembC, _DN_00) + bcC
        WrP = dot(Wr, Wp)
        WrF = dot(Wemb, WrP)
        brF = dg(WrP, bembC, _DN_00) + dg(Wp, brC, _DN_00)
        eW, eB = [], []
        for e in range(E):
            We_e = W_ref[24 + 16 * e:40 + 16 * e, :]
            beC_e = W_ref[216 + 16 * e:232 + 16 * e, 0:1]
            WeP = dot(We_e, Wp)
            eW.append(dot(Wemb, WeP))
            eB.append(dg(WeP, bembC, _DN_00) + dg(Wp, beC_e, _DN_00))
        WF_s[...] = jnp.concatenate(eW + [WrF, WcF], axis=1)
        bias_s[...] = jnp.concatenate(eB + [brF, bcF], axis=0)

    WF = WF_s[...]
    biasC = bias_s[...]

    x = x_ref[...]
    hT = dg(Wemb, x, _DN_01) + bembC
    logits = dg(Wg, hT, _DN_00)
    m = jnp.max(logits, axis=0, keepdims=True)
    ex = jnp.exp(logits - m)
    denom = jnp.sum(ex, axis=0, keepdims=True)
    gate = 1.0 / denom
    iota = lax.broadcasted_iota(jnp.int32, logits.shape, 0)
    idx = jnp.min(jnp.where(logits >= m, iota, E), axis=0, keepdims=True)

    A = dg(WF, x, _DN_01) + biasC
    iota32 = lax.broadcasted_iota(jnp.int32, (E * D_OUT, x.shape[0]), 0)
    mask32 = (lax.div(iota32, 4) == idx).astype(f32)
    masked = A[0:32, :] * mask32
    s1 = masked[0:16, :] + masked[16:32, :]
    s2 = s1[0:8, :] + s1[8:16, :]
    acc = s2[0:4, :] + s2[4:8, :]

    c0 = jax.nn.sigmoid(A[36:37, :] - A[37:38, :])
    outF = (c0 * gate) * acc + (1.0 - c0) * A[32:36, :]
    out_ref[...] = dg(outF, jnp.eye(4, dtype=f32), _DN_00) + bp


def _pack_weights(W_embed, b_embed, Wg, We, be, Wr, br, Wc, bc, Wp, bp):
    def sec(arr, rows):
        r, c = arr.shape
        return jnp.pad(arr, ((0, rows - r), (0, 16 - c)))
    return jnp.concatenate([
        sec(W_embed, 8),
        sec(Wg, 16),
        We.reshape(E * D_MODEL, D_MODEL),
        sec(Wr, 16),
        sec(Wc, 16),
        sec(Wp, 16),
        sec(b_embed.reshape(-1, 1), 16),
        sec(be.reshape(-1, 1), 128),
        sec(br.reshape(-1, 1), 16),
        sec(bc.reshape(-1, 1), 8),
        sec(bp.reshape(1, -1), 8),
    ], axis=0)


@jax.jit
def kernel(x, W_embed, b_embed, Wg, We, be, Wr, br, Wc, bc, Wp, bp):
    W_all = _pack_weights(W_embed, b_embed, Wg, We, be, Wr, br, Wc, bc,
                          Wp, bp)
    out = pl.pallas_call(
        _moe_kernel,
        grid=(N // BLOCK,),
        in_specs=[
            pl.BlockSpec((BLOCK, D_IN), lambda i: (i, 0)),
            pl.BlockSpec((W_ROWS, 16), lambda i: (0, 0)),
        ],
        out_specs=pl.BlockSpec((BLOCK, D_OUT), lambda i: (i, 0)),
        out_shape=jax.ShapeDtypeStruct((N, D_OUT), jnp.float32),
        scratch_shapes=[pltpu.VMEM((D_IN, 38), jnp.float32),
                        pltpu.VMEM((38, 1), jnp.float32)],
        compiler_params=pltpu.CompilerParams(
            dimension_semantics=("parallel",)),
    )(x, W_all)
    return out


# speedup vs baseline: 1.0026x; 1.0026x over previous
import jax
import jax.numpy as jnp
from jax import lax
from jax.experimental import pallas as pl
from jax.experimental.pallas import tpu as pltpu

N = 32768
D_IN, D_MODEL, E, D_OUT = 4, 16, 8, 4
BLOCK = 8192
W_ROWS = 376

_DN_00 = (((0,), (0,)), ((), ()))
_DN_01 = (((0,), (1,)), ((), ()))


def _moe_kernel(x_ref, W_ref, out_ref, WF_s, bias_s):
    f32 = jnp.float32
    dot = lambda a, b: jnp.dot(a, b, preferred_element_type=f32)
    dg = lambda a, b, dn: lax.dot_general(a, b, dn, preferred_element_type=f32)

    Wemb = W_ref[0:4, :]
    Wg = W_ref[8:24, 0:8]
    Wr = W_ref[152:168, :]
    Wc = W_ref[168:184, 0:2]
    Wp = W_ref[184:200, 0:4]
    bembC = W_ref[200:216, 0:1]
    brC = W_ref[344:360, 0:1]
    bcC = W_ref[360:362, 0:1]
    bp = W_ref[368:369, 0:4]

    @pl.when(pl.program_id(0) == 0)
    def _fold():
        WcF = dot(Wemb, Wc)
        bcF = dg(Wc, bembC, _DN_00) + bcC
        WrP = dot(Wr, Wp)
        WrF = dot(Wemb, WrP)
        brF = dg(WrP, bembC, _DN_00) + dg(Wp, brC, _DN_00)
        eW, eB = [], []
        for e in range(E):
            We_e = W_ref[24 + 16 * e:40 + 16 * e, :]
            beC_e = W_ref[216 + 16 * e:232 + 16 * e, 0:1]
            WeP = dot(We_e, Wp)
            eW.append(dot(Wemb, WeP))
            eB.append(dg(WeP, bembC, _DN_00) + dg(Wp, beC_e, _DN_00))
        WF_s[...] = jnp.concatenate(eW + [WrF, WcF], axis=1)
        bias_s[...] = jnp.concatenate(eB + [brF, bcF], axis=0)

    WF = WF_s[...]
    biasC = bias_s[...]

    x = x_ref[...]
    hT = dg(Wemb, x, _DN_01) + bembC
    logits = dg(Wg, hT, _DN_00)
    m = jnp.max(logits, axis=0, keepdims=True)
    ex = jnp.exp(logits - m)
    denom = jnp.sum(ex, axis=0, keepdims=True)
    gate = 1.0 / denom
    iota = lax.broadcasted_iota(jnp.int32, logits.shape, 0)
    idx = jnp.min(jnp.where(logits >= m, iota, E), axis=0, keepdims=True)

    A = dg(WF, x, _DN_01) + biasC
    iota32 = lax.broadcasted_iota(jnp.int32, (E * D_OUT, x.shape[0]), 0)
    mask32 = (lax.div(iota32, 4) == idx).astype(f32)
    masked = A[0:32, :] * mask32
    s1 = masked[0:16, :] + masked[16:32, :]
    s2 = s1[0:8, :] + s1[8:16, :]
    acc = s2[0:4, :] + s2[4:8, :]

    c0 = jax.nn.sigmoid(A[36:37, :] - A[37:38, :])
    outF = (c0 * gate) * acc + (1.0 - c0) * A[32:36, :]
    out_ref[...] = dg(outF, jnp.eye(4, dtype=f32), _DN_00) + bp


def _pack_weights(W_embed, b_embed, Wg, We, be, Wr, br, Wc, bc, Wp, bp):
    def sec(arr, rows):
        r, c = arr.shape
        return jnp.pad(arr, ((0, rows - r), (0, 16 - c)))
    return jnp.concatenate([
        sec(W_embed, 8),
        sec(Wg, 16),
        We.reshape(E * D_MODEL, D_MODEL),
        sec(Wr, 16),
        sec(Wc, 16),
        sec(Wp, 16),
        sec(b_embed.reshape(-1, 1), 16),
        sec(be.reshape(-1, 1), 128),
        sec(br.reshape(-1, 1), 16),
        sec(bc.reshape(-1, 1), 8),
        sec(bp.reshape(1, -1), 8),
    ], axis=0)


@jax.jit
def kernel(x, W_embed, b_embed, Wg, We, be, Wr, br, Wc, bc, Wp, bp):
    W_all = _pack_weights(W_embed, b_embed, Wg, We, be, Wr, br, Wc, bc,
                          Wp, bp)
    out = pl.pallas_call(
        _moe_kernel,
        grid=(N // BLOCK,),
        in_specs=[
            pl.BlockSpec((BLOCK, D_IN), lambda i: (i, 0)),
            pl.BlockSpec((W_ROWS, 16), lambda i: (0, 0)),
        ],
        out_specs=pl.BlockSpec((BLOCK, D_OUT), lambda i: (i, 0)),
        out_shape=jax.ShapeDtypeStruct((N, D_OUT), jnp.float32),
        scratch_shapes=[pltpu.VMEM((D_IN, 38), jnp.float32),
                        pltpu.VMEM((38, 1), jnp.float32)],
    )(x, W_all)
    return out
